# baseline (speedup 1.0000x reference)
import functools

import jax
import jax.numpy as jnp
from jax import lax
from jax.experimental import pallas as pl
from jax.experimental.pallas import tpu as pltpu
from jax.experimental.pallas import tpu_sc as plsc

VOCAB = 1000000
HID = 64
B = 16384
L = 50
POS_ROWS = 128

NC = 2
NS = 16
NW = NC * NS

BB = 128
NBB = B // BB
BB_PW = NBB // NW
NCHUNK = BB_PW * L
POS_COPY = 56


def _sc_body(idxt_hbm, w_hbm, pos_hbm, out5_hbm, idx_v, wide, sbuf, pos_v,
             gsem0, gsem1, osem0, osem1):
    wid = lax.axis_index("s") * NC + lax.axis_index("c")
    pltpu.sync_copy(pos_hbm.at[pl.ds(0, POS_COPY)], pos_v)
    gsems = (gsem0, gsem1)
    osems = (osem0, osem1)
    iota16 = lax.iota(jnp.int32, 16)

    def lbb_of(c):
        lc = lax.rem(c, L)
        bbg = wid * BB_PW + c // L
        return lc, bbg

    def load_idx(c, s):
        lc, bbg = lbb_of(c)
        b0 = pl.multiple_of(bbg * BB, BB)
        pltpu.sync_copy(idxt_hbm.at[pl.ds(lc, 1)].at[:, pl.ds(b0, BB)],
                        idx_v.at[s])

    def gather_copy(s):
        return pltpu.make_async_copy(
            w_hbm.at[idx_v.at[s].at[0]], wide.at[s], gsems[s])

    def store_copy(c, s):
        lc, bbg = lbb_of(c)
        return pltpu.make_async_copy(
            sbuf.at[s],
            out5_hbm.at[pl.ds(lc, 1), :, pl.ds(bbg, 1), :, :], osems[s])

    def wait_store(s):
        pltpu.make_async_copy(
            sbuf.at[s],
            out5_hbm.at[pl.ds(0, 1), :, pl.ds(0, 1), :, :], osems[s]).wait()

    def shuffle_add(c, s):
        lc, _ = lbb_of(c)

        def hb_body(hb, carry):
            for hi in range(8):
                h = hb * 8 + hi
                colv = jnp.full((16,), 0, jnp.int32) + h
                pv = plsc.load_gather(pos_v,
                                      [jnp.full((16,), 0, jnp.int32) + lc,
                                       colv])
                for g in range(BB // 16):
                    rowv = iota16 + (g * 16)
                    vals = plsc.load_gather(wide.at[s], [rowv, colv])
                    sbuf[s, 0, hb, 0, hi, pl.ds(g * 16, 16)] = vals + pv
            return carry

        lax.fori_loop(0, 8, hb_body, 0)

    load_idx(0, 0)
    gather_copy(0).start()

    @pl.loop(0, NCHUNK, step=2)
    def _chunks(c0):
        for b in range(2):
            c = c0 + b
            nxt = c + 1

            @pl.when(nxt < NCHUNK)
            def _prefetch():
                load_idx(nxt, 1 - b)

                @pl.when(c >= 1)
                def _drain_prev_store():
                    wait_store(1 - b)

                gather_copy(1 - b).start()

            gather_copy(b).wait()
            shuffle_add(c, b)
            store_copy(c, b).start()

    wait_store(0)
    wait_store(1)


@jax.jit
def _phrase_embedding_sc(idxt, w, pos):
    mesh = plsc.VectorSubcoreMesh(
        core_axis_name="c", subcore_axis_name="s",
        num_cores=NC, num_subcores=NS)
    call = functools.partial(
        pl.kernel,
        out_type=jax.ShapeDtypeStruct((L, HID // 8, NBB, 8, BB), jnp.float32),
        mesh=mesh,
        scratch_types=[
            pltpu.VMEM((2, 1, BB), jnp.int32),
            pltpu.VMEM((2, BB, HID), jnp.float32),
            pltpu.VMEM((2, 1, HID // 8, 1, 8, BB), jnp.float32),
            pltpu.VMEM((POS_COPY, HID), jnp.float32),
            pltpu.SemaphoreType.DMA,
            pltpu.SemaphoreType.DMA,
            pltpu.SemaphoreType.DMA,
            pltpu.SemaphoreType.DMA,
        ],
        compiler_params=pltpu.CompilerParams(use_tc_tiling_on_sc=False,
                                             needs_layout_passes=False),
    )(_sc_body)
    return call(idxt, w, pos)


def kernel(phrase, W, pos_emb):
    idxt = phrase.astype(jnp.int32).T
    out5 = _phrase_embedding_sc(idxt, W, pos_emb)
    return out5.transpose(2, 4, 0, 1, 3).reshape(B, L, HID)


# speedup vs baseline: 1.1508x; 1.1508x over previous
import functools

import jax
import jax.numpy as jnp
from jax import lax
from jax.experimental import pallas as pl
from jax.experimental.pallas import tpu as pltpu
from jax.experimental.pallas import tpu_sc as plsc

VOCAB = 1000000
HID = 64
B = 16384
L = 50
POS_ROWS = 128

NC = 2
NS = 16
NW = NC * NS

BB = 128
NBB = B // BB
BB_PW = NBB // NW
VECS = HID // 16
SLAB = 8 * 8 * BB
POS_COPY = 56


def _sc_body(idxt_hbm, w_hbm, pos_hbm, out_hbm, idx_v, wide, sbuf, pos_v,
             gsem0, gsem1, osem0, osem1):
    wid = lax.axis_index("s") * NC + lax.axis_index("c")
    pltpu.sync_copy(pos_hbm.at[pl.ds(0, POS_COPY)], pos_v)
    gsems = (gsem0, gsem1)
    osems = (osem0, osem1)

    iota = lax.iota(jnp.int32, 16)
    scat_base = [((2 * q + (iota >> 3)) << 10) + ((iota & 7) << 7)
                 for q in range(VECS)]

    def gather_copy(lc, s):
        return pltpu.make_async_copy(
            w_hbm.at[idx_v.at[lc]], wide.at[s], gsems[s])

    def store_copies(lc, bbx, s):
        cps = []
        for hb in range(8):
            row = (lc * 8 + hb) * NBB + bbx
            cps.append(pltpu.make_async_copy(
                sbuf.at[s].at[pl.ds(hb * 1024, 1024)],
                out_hbm.at[pl.ds(pl.multiple_of(row * 1024, 1024), 1024)],
                osems[s]))
        return cps

    def wait_store(s):
        for hb in range(8):
            pltpu.make_async_copy(
                sbuf.at[s].at[pl.ds(hb * 1024, 1024)],
                out_hbm.at[pl.ds(hb * 1024, 1024)], osems[s]).wait()

    def shuffle_add(lc, s):
        posv = [pos_v[lc, pl.ds(q * 16, 16)] for q in range(VECS)]

        @pl.loop(0, BB, unroll=4)
        def _bi(bi):
            bivec = jnp.full((16,), 0, jnp.int32) + bi
            for q in range(VECS):
                v = wide[s, bi, pl.ds(q * 16, 16)] + posv[q]
                plsc.store_scatter(sbuf.at[s], [scat_base[q] + bivec], v)

    for bbi in range(BB_PW):
        bbx = wid * BB_PW + bbi
        b0 = pl.multiple_of(bbx * BB, BB)
        pltpu.sync_copy(idxt_hbm.at[:, pl.ds(b0, BB)], idx_v)

        gather_copy(0, 0).start()

        @pl.loop(0, L, step=2)
        def _chunks(c0):
            for b in range(2):
                c = c0 + b
                nxt = c + 1

                @pl.when(nxt < L)
                def _prefetch():
                    @pl.when(c >= 1)
                    def _drain_prev_store():
                        wait_store(1 - b)

                    gather_copy(nxt, 1 - b).start()

                gather_copy(c, b).wait()
                shuffle_add(c, b)
                for cp in store_copies(c, bbx, b):
                    cp.start()

        wait_store(0)
        wait_store(1)


@jax.jit
def _phrase_embedding_sc(idxt, w, pos):
    mesh = plsc.VectorSubcoreMesh(
        core_axis_name="c", subcore_axis_name="s",
        num_cores=NC, num_subcores=NS)
    call = functools.partial(
        pl.kernel,
        out_type=jax.ShapeDtypeStruct((B * L * HID,), jnp.float32),
        mesh=mesh,
        scratch_types=[
            pltpu.VMEM((L, BB), jnp.int32),
            pltpu.VMEM((2, BB, HID), jnp.float32),
            pltpu.VMEM((2, SLAB), jnp.float32),
            pltpu.VMEM((POS_COPY, HID), jnp.float32),
            pltpu.SemaphoreType.DMA,
            pltpu.SemaphoreType.DMA,
            pltpu.SemaphoreType.DMA,
            pltpu.SemaphoreType.DMA,
        ],
        compiler_params=pltpu.CompilerParams(use_tc_tiling_on_sc=False,
                                             needs_layout_passes=False),
    )(_sc_body)
    return call(idxt, w, pos)


def kernel(phrase, W, pos_emb):
    idxt = phrase.astype(jnp.int32).T
    out = _phrase_embedding_sc(idxt, W, pos_emb)
    return (out.reshape(L, HID // 8, NBB, 8, BB)
               .transpose(2, 4, 0, 1, 3).reshape(B, L, HID))


# speedup vs baseline: 1.4900x; 1.2947x over previous
import functools

import jax
import jax.numpy as jnp
from jax import lax
from jax.experimental import pallas as pl
from jax.experimental.pallas import tpu as pltpu
from jax.experimental.pallas import tpu_sc as plsc

VOCAB = 1000000
HID = 64
B = 16384
L = 50
POS_ROWS = 128

NC = 2
NS = 16
NW = NC * NS

BB = 128
NBB = B // BB
BB_PW = NBB // NW
VECS = HID // 16
SLAB = 8 * 8 * BB
POS_COPY = 56


def _sc_body(idxt_hbm, w_hbm, pos_hbm, out_hbm, idx_v, wide, sbuf, pos_v,
             gsem0, gsem1, osem0, osem1):
    wid = lax.axis_index("s") * NC + lax.axis_index("c")
    pltpu.sync_copy(pos_hbm.at[pl.ds(0, POS_COPY)], pos_v)
    gsems = (gsem0, gsem1)
    osems = (osem0, osem1)

    iota = lax.iota(jnp.int32, 16)
    scat_base = [((2 * q + (iota >> 3)) << 10) + ((iota & 7) << 7)
                 for q in range(VECS)]

    def gather_copy(lc, s):
        return pltpu.make_async_copy(
            w_hbm.at[idx_v.at[lc]], wide.at[s], gsems[s])

    def store_copies(lc, bbx, s):
        cps = []
        for hb in range(8):
            row = (lc * 8 + hb) * NBB + bbx
            cps.append(pltpu.make_async_copy(
                sbuf.at[s].at[pl.ds(hb * 1024, 1024)],
                out_hbm.at[pl.ds(pl.multiple_of(row * 1024, 1024), 1024)],
                osems[s]))
        return cps

    def wait_store(s):
        for hb in range(8):
            pltpu.make_async_copy(
                sbuf.at[s].at[pl.ds(hb * 1024, 1024)],
                out_hbm.at[pl.ds(hb * 1024, 1024)], osems[s]).wait()

    def shuffle_add(lc, s):
        posv = [pos_v[lc, pl.ds(q * 16, 16)] for q in range(VECS)]

        @plsc.parallel_loop(0, BB, step=1, unroll=4)
        def _bi(bi):
            bivec = jnp.full((16,), 0, jnp.int32) + bi
            for q in range(VECS):
                v = wide[s, bi, pl.ds(q * 16, 16)] + posv[q]
                plsc.store_scatter(sbuf.at[s], [scat_base[q] + bivec], v)

    for bbi in range(BB_PW):
        bbx = wid * BB_PW + bbi
        b0 = pl.multiple_of(bbx * BB, BB)
        pltpu.sync_copy(idxt_hbm.at[:, pl.ds(b0, BB)], idx_v)

        gather_copy(0, 0).start()

        @pl.loop(0, L, step=2)
        def _chunks(c0):
            for b in range(2):
                c = c0 + b
                nxt = c + 1

                @pl.when(nxt < L)
                def _prefetch():
                    @pl.when(c >= 1)
                    def _drain_prev_store():
                        wait_store(1 - b)

                    gather_copy(nxt, 1 - b).start()

                gather_copy(c, b).wait()
                shuffle_add(c, b)
                for cp in store_copies(c, bbx, b):
                    cp.start()

        wait_store(0)
        wait_store(1)


@jax.jit
def _phrase_embedding_sc(idxt, w, pos):
    mesh = plsc.VectorSubcoreMesh(
        core_axis_name="c", subcore_axis_name="s",
        num_cores=NC, num_subcores=NS)
    call = functools.partial(
        pl.kernel,
        out_type=jax.ShapeDtypeStruct((B * L * HID,), jnp.float32),
        mesh=mesh,
        scratch_types=[
            pltpu.VMEM((L, BB), jnp.int32),
            pltpu.VMEM((2, BB, HID), jnp.float32),
            pltpu.VMEM((2, SLAB), jnp.float32),
            pltpu.VMEM((POS_COPY, HID), jnp.float32),
            pltpu.SemaphoreType.DMA,
            pltpu.SemaphoreType.DMA,
            pltpu.SemaphoreType.DMA,
            pltpu.SemaphoreType.DMA,
        ],
        compiler_params=pltpu.CompilerParams(use_tc_tiling_on_sc=False,
                                             needs_layout_passes=False),
    )(_sc_body)
    return call(idxt, w, pos)


def kernel(phrase, W, pos_emb):
    idxt = phrase.astype(jnp.int32).T
    out = _phrase_embedding_sc(idxt, W, pos_emb)
    return (out.reshape(L, HID // 8, NBB, 8, BB)
               .transpose(2, 4, 0, 1, 3).reshape(B, L, HID))


# speedup vs baseline: 2.5552x; 1.7149x over previous
import functools

import jax
import jax.numpy as jnp
from jax import lax
from jax.experimental import pallas as pl
from jax.experimental.pallas import tpu as pltpu
from jax.experimental.pallas import tpu_sc as plsc

VOCAB = 1000000
HID = 64
B = 16384
L = 50
POS_ROWS = 128

NC = 2
NS = 16
NW = NC * NS

BB = 128
NBB = B // BB
BB_PW = NBB // NW
VECS = HID // 16
SLAB = 8 * 8 * BB
POS_COPY = 56


def _sc_body(idxt_hbm, w_hbm, pos_hbm, out_hbm, idx_v, wide, sbuf, pos_v,
             gsem0, gsem1, osem0, osem1):
    wid = lax.axis_index("s") * NC + lax.axis_index("c")
    pltpu.sync_copy(pos_hbm.at[pl.ds(0, POS_COPY)], pos_v)
    gsems = (gsem0, gsem1)
    osems = (osem0, osem1)

    iota = lax.iota(jnp.int32, 16)
    hbv = [(2 * q) + (iota >> 3) for q in range(VECS)]
    hiv = iota & 7

    def gather_copy(lc, s):
        return pltpu.make_async_copy(
            w_hbm.at[idx_v.at[lc]], wide.at[s], gsems[s])

    def store_copy(lc, bbx, s):
        return pltpu.make_async_copy(
            sbuf.at[s].at[:, :, pl.ds(0, BB)],
            out_hbm.at[lc, :, bbx, :, :], osems[s])

    def wait_store(s):
        pltpu.make_async_copy(
            sbuf.at[s].at[:, :, pl.ds(0, BB)],
            out_hbm.at[0, :, 0, :, :], osems[s]).wait()

    def shuffle_add(lc, s):
        posv = [pos_v[lc, pl.ds(q * 16, 16)] for q in range(VECS)]

        @plsc.parallel_loop(0, BB, step=1, unroll=4)
        def _bi(bi):
            bivec = jnp.full((16,), 0, jnp.int32) + bi
            for q in range(VECS):
                v = wide[s, bi, pl.ds(q * 16, 16)] + posv[q]
                plsc.store_scatter(sbuf.at[s], [hbv[q], hiv, bivec], v)

    for bbi in range(BB_PW):
        bbx = wid * BB_PW + bbi
        b0 = pl.multiple_of(bbx * BB, BB)
        pltpu.sync_copy(idxt_hbm.at[:, pl.ds(b0, BB)], idx_v)

        gather_copy(0, 0).start()

        @pl.loop(0, L, step=2)
        def _chunks(c0):
            for b in range(2):
                c = c0 + b
                nxt = c + 1

                @pl.when(nxt < L)
                def _prefetch():
                    @pl.when(c >= 1)
                    def _drain_prev_store():
                        wait_store(1 - b)

                    gather_copy(nxt, 1 - b).start()

                gather_copy(c, b).wait()
                shuffle_add(c, b)
                store_copy(c, bbx, b).start()

        wait_store(0)
        wait_store(1)


@jax.jit
def _phrase_embedding_sc(idxt, w, pos):
    mesh = plsc.VectorSubcoreMesh(
        core_axis_name="c", subcore_axis_name="s",
        num_cores=NC, num_subcores=NS)
    call = functools.partial(
        pl.kernel,
        out_type=jax.ShapeDtypeStruct((L, HID // 8, NBB, 8, BB), jnp.float32),
        mesh=mesh,
        scratch_types=[
            pltpu.VMEM((L, BB), jnp.int32),
            pltpu.VMEM((2, BB, HID), jnp.float32),
            pltpu.VMEM((2, 8, 8, BB + 1), jnp.float32),
            pltpu.VMEM((POS_COPY, HID), jnp.float32),
            pltpu.SemaphoreType.DMA,
            pltpu.SemaphoreType.DMA,
            pltpu.SemaphoreType.DMA,
            pltpu.SemaphoreType.DMA,
        ],
        compiler_params=pltpu.CompilerParams(use_tc_tiling_on_sc=False,
                                             needs_layout_passes=False),
    )(_sc_body)
    return call(idxt, w, pos)


def kernel(phrase, W, pos_emb):
    idxt = phrase.astype(jnp.int32).T
    out5 = _phrase_embedding_sc(idxt, W, pos_emb)
    return out5.transpose(2, 4, 0, 1, 3).reshape(B, L, HID)
